# baseline (speedup 1.0000x reference)
import functools

import numpy as np
import jax
import jax.numpy as jnp
from jax import lax
from jax.experimental import pallas as pl
from jax.experimental.pallas import tpu as pltpu
from jax.experimental.pallas import tpu_sc as plsc

_B = 256
_C = 3
_H = 224
_Wd = 224
_RH = 18
_RW = 18
_K = 9
_OH = 16
_OW = 16
_NW = 32
_BPW = _B // _NW
_PLANE = _RH * _RW
_IMG = _C * _PLANE
_OFFS = 2 * _K * _OH * _OW
_MSKS = _K * _OH * _OW
_OUTS = _OH * _OW
_BS = 8


def _interp_matrix(out_n, in_n):
    s = in_n / out_n
    pos = np.maximum((np.arange(out_n) + 0.5) * s - 0.5, 0.0).astype(np.float32)
    i0 = np.floor(pos).astype(np.int32)
    i1 = np.minimum(i0 + 1, in_n - 1)
    f = (pos - i0).astype(np.float32)
    m = np.zeros((out_n, in_n), np.float32)
    m[np.arange(out_n), i0] += 1.0 - f
    m[np.arange(out_n), i1] += f
    return m


def _row_plan():
    s = _H / _RH
    pos = np.maximum((np.arange(_RH) + 0.5) * s - 0.5, 0.0)
    y0 = np.floor(pos).astype(np.int32)
    y1 = np.minimum(y0 + 1, _H - 1)
    f = (pos - y0).astype(np.float32)
    rows = np.stack([y0, y1], axis=1).reshape(-1)
    wts = np.stack([1.0 - f, f], axis=1).reshape(-1).astype(np.float32)
    return rows, wts


def _resize_body(rows_ref, w_ref, x_ref, rxt_ref, o_ref):
    j = pl.program_id(0)
    w = w_ref[j]
    cb = jnp.dot(x_ref[...].reshape(_B * _C, _Wd), rxt_ref[...],
                 preferred_element_type=jnp.float32)
    contrib = (w * cb).reshape(1, _B, _C, _RW)

    @pl.when(j % 2 == 0)
    def _():
        o_ref[...] = contrib

    @pl.when(j % 2 == 1)
    def _():
        o_ref[...] += contrib


def _resize(x):
    rows, wts = _row_plan()
    rxt = jnp.asarray(_interp_matrix(_RW, _Wd).T)
    out = pl.pallas_call(
        _resize_body,
        grid_spec=pltpu.PrefetchScalarGridSpec(
            num_scalar_prefetch=2,
            grid=(2 * _RH,),
            in_specs=[
                pl.BlockSpec((_B, _C, 1, 1, _Wd),
                             lambda j, rows_ref, w_ref: (0, 0, rows_ref[j], 0, 0)),
                pl.BlockSpec((_Wd, _RW), lambda j, rows_ref, w_ref: (0, 0)),
            ],
            out_specs=pl.BlockSpec(
                (1, _B, _C, _RW), lambda j, rows_ref, w_ref: (j // 2, 0, 0, 0)),
        ),
        out_shape=jax.ShapeDtypeStruct((_RH, _B, _C, _RW), jnp.float32),
        compiler_params=pltpu.CompilerParams(
            dimension_semantics=("arbitrary",)),
    )(jnp.asarray(rows), jnp.asarray(wts),
      x.reshape(_B, _C, _H, 1, _Wd), rxt)
    return jnp.transpose(out, (1, 2, 0, 3))


def _deform_sc(inp_flat, off_flat, mask_flat, w_pad):
    mesh = plsc.VectorSubcoreMesh(core_axis_name="c", subcore_axis_name="s")

    @functools.partial(
        pl.kernel,
        mesh=mesh,
        compiler_params=pltpu.CompilerParams(needs_layout_passes=False),
        out_type=jax.ShapeDtypeStruct((_B * _OUTS,), jnp.float32),
        scratch_types=[
            pltpu.VMEM((_BPW * _IMG,), jnp.float32),
            pltpu.VMEM((_BPW * _OFFS,), jnp.float32),
            pltpu.VMEM((_BPW * _MSKS,), jnp.float32),
            pltpu.VMEM((32,), jnp.float32),
            pltpu.VMEM((_BPW * _OUTS,), jnp.float32),
        ],
    )
    def _deform(inp_hbm, off_hbm, mask_hbm, w_hbm, out_hbm,
                inp_v, off_v, mask_v, w_v, out_v):
        wid = lax.axis_index("s") * 2 + lax.axis_index("c")
        pltpu.sync_copy(inp_hbm.at[pl.ds(wid * (_BPW * _IMG), _BPW * _IMG)], inp_v)
        pltpu.sync_copy(off_hbm.at[pl.ds(wid * (_BPW * _OFFS), _BPW * _OFFS)], off_v)
        pltpu.sync_copy(mask_hbm.at[pl.ds(wid * (_BPW * _MSKS), _BPW * _MSKS)], mask_v)
        pltpu.sync_copy(w_hbm, w_v)

        lanes_f = lax.iota(jnp.int32, 16).astype(jnp.float32)
        zero16 = jnp.zeros((16,), jnp.float32)
        one16 = jnp.ones((16,), jnp.float32)

        def body_b(i, c0):
            def body_v(v, c1):
                def body_k(k, acc):
                    ki = k // 3
                    kj = k % 3
                    obase = i * _OFFS + 2 * k * _OUTS + v * _OW
                    dy = off_v[pl.ds(obase, 16)]
                    dx = off_v[pl.ds(obase + _OUTS, 16)]
                    m = mask_v[pl.ds(i * _MSKS + k * _OUTS + v * _OW, 16)]
                    yy = dy + (v + ki).astype(jnp.float32)
                    xx = dx + kj.astype(jnp.float32) + lanes_f
                    ty = yy.astype(jnp.int32)
                    y0 = ty - jnp.where(ty.astype(jnp.float32) > yy, 1, 0)
                    fy = yy - y0.astype(jnp.float32)
                    tx = xx.astype(jnp.int32)
                    x0 = tx - jnp.where(tx.astype(jnp.float32) > xx, 1, 0)
                    fx = xx - x0.astype(jnp.float32)
                    y1 = y0 + 1
                    x1 = x0 + 1
                    vy0 = jnp.where((y0 >= 0) & (y0 < _RH), one16, zero16)
                    vy1 = jnp.where((y1 >= 0) & (y1 < _RH), one16, zero16)
                    vx0 = jnp.where((x0 >= 0) & (x0 < _RW), one16, zero16)
                    vx1 = jnp.where((x1 >= 0) & (x1 < _RW), one16, zero16)
                    cy0 = jnp.clip(y0, 0, _RH - 1)
                    cy1 = jnp.clip(y1, 0, _RH - 1)
                    cx0 = jnp.clip(x0, 0, _RW - 1)
                    cx1 = jnp.clip(x1, 0, _RW - 1)
                    gy0 = vy0 * (1.0 - fy)
                    gy1 = vy1 * fy
                    gx0 = vx0 * (1.0 - fx)
                    gx1 = vx1 * fx
                    w00 = gy0 * gx0
                    w01 = gy0 * gx1
                    w10 = gy1 * gx0
                    w11 = gy1 * gx1
                    r0 = cy0 * _RW
                    r1 = cy1 * _RW
                    i00 = r0 + cx0
                    i01 = r0 + cx1
                    i10 = r1 + cx0
                    i11 = r1 + cx1
                    ib = i * _IMG
                    tot = zero16
                    for c in range(_C):
                        base = ib + c * _PLANE
                        v00 = plsc.load_gather(inp_v, [i00 + base])
                        v01 = plsc.load_gather(inp_v, [i01 + base])
                        v10 = plsc.load_gather(inp_v, [i10 + base])
                        v11 = plsc.load_gather(inp_v, [i11 + base])
                        s = w00 * v00 + w01 * v01 + w10 * v10 + w11 * v11
                        wk = plsc.load_gather(
                            w_v, [jnp.full((16,), c * _K, jnp.int32) + k])
                        tot = tot + wk * s
                    return acc + m * tot

                acc = lax.fori_loop(0, _K, body_k, zero16)
                out_v[pl.ds(i * _OUTS + v * _OW, 16)] = acc
                return c1

            lax.fori_loop(0, _OH, body_v, 0)
            return c0

        lax.fori_loop(0, _BPW, body_b, 0)
        pltpu.sync_copy(out_v, out_hbm.at[pl.ds(wid * (_BPW * _OUTS), _BPW * _OUTS)])

    return _deform(inp_flat, off_flat, mask_flat, w_pad)


def kernel(x, offset, mask, W):
    inp = _resize(x)
    out_flat = _deform_sc(
        inp.reshape(_B * _IMG),
        offset.reshape(_B * _OFFS),
        mask.reshape(_B * _MSKS),
        jnp.pad(W.reshape(_C * _K), (0, 32 - _C * _K)),
    )
    return out_flat.reshape(_B, 1, _OH, _OW)


# speedup vs baseline: 2.0886x; 2.0886x over previous
import functools

import numpy as np
import jax
import jax.numpy as jnp
from jax import lax
from jax.experimental import pallas as pl
from jax.experimental.pallas import tpu as pltpu
from jax.experimental.pallas import tpu_sc as plsc

_B = 256
_C = 3
_H = 224
_Wd = 224
_RH = 18
_RW = 18
_K = 9
_OH = 16
_OW = 16
_NW = 32
_BPW = _B // _NW
_RSTRIDE = 32
_PLANE = _RH * _RSTRIDE
_IMG = _C * _PLANE
_NROW = 2 * _RH * _C
_NROWP = 112
_OFFS = 2 * _K * _OH * _OW
_MSKS = _K * _OH * _OW
_OUTS = _OH * _OW


def _interp_plan(out_n, in_n):
    s = in_n / out_n
    pos = np.maximum((np.arange(out_n) + 0.5) * s - 0.5, 0.0)
    i0 = np.floor(pos).astype(np.int32)
    i1 = np.minimum(i0 + 1, in_n - 1)
    f = (pos - i0).astype(np.float32)
    return i0, i1, f


def _consts():
    _, _, fy = _interp_plan(_RH, _H)
    rw = np.zeros((64,), np.float32)
    rw[:_RH] = 1.0 - fy
    rw[32:32 + _RH] = fy
    x0, x1, fx = _interp_plan(_RW, _Wd)
    coli = np.zeros((64,), np.int32)
    colw = np.zeros((64,), np.float32)
    coli[:_RW] = x0
    coli[32:32 + _RW] = x1
    colw[:_RW] = 1.0 - fx
    colw[32:32 + _RW] = fx
    return rw, coli, colw


def _fused_sc(xsel, off_flat, mask_flat, w_pad, rw, coli, colw):
    mesh = plsc.VectorSubcoreMesh(core_axis_name="c", subcore_axis_name="s")

    @functools.partial(
        pl.kernel,
        mesh=mesh,
        compiler_params=pltpu.CompilerParams(needs_layout_passes=False),
        out_type=jax.ShapeDtypeStruct((_B * _OUTS,), jnp.float32),
        scratch_types=[
            pltpu.VMEM((_C * _RH, 2 * _Wd), jnp.float32),
            pltpu.VMEM((_Wd,), jnp.float32),
            pltpu.VMEM((_IMG,), jnp.float32),
            pltpu.VMEM((_BPW * _OFFS,), jnp.float32),
            pltpu.VMEM((_BPW * _MSKS,), jnp.float32),
            pltpu.VMEM((32,), jnp.float32),
            pltpu.VMEM((64,), jnp.float32),
            pltpu.VMEM((64,), jnp.int32),
            pltpu.VMEM((64,), jnp.float32),
            pltpu.VMEM((_BPW * _OUTS,), jnp.float32),
            pltpu.SemaphoreType.DMA,
        ],
    )
    def _fused(x_hbm, off_hbm, mask_hbm, w_hbm, rw_hbm,
               coli_hbm, colw_hbm, out_hbm,
               rows_v, ri_v, inp_v, off_v, mask_v, w_v,
               rw_v, coli_v, colw_v, out_v, sem):
        wid = lax.axis_index("s") * 2 + lax.axis_index("c")
        pltpu.sync_copy(off_hbm.at[pl.ds(wid * (_BPW * _OFFS), _BPW * _OFFS)], off_v)
        pltpu.sync_copy(mask_hbm.at[pl.ds(wid * (_BPW * _MSKS), _BPW * _MSKS)], mask_v)
        pltpu.sync_copy(w_hbm, w_v)
        pltpu.sync_copy(rw_hbm, rw_v)
        pltpu.sync_copy(coli_hbm, coli_v)
        pltpu.sync_copy(colw_hbm, colw_v)

        lanes_i = lax.iota(jnp.int32, 16)
        lanes_f = lanes_i.astype(jnp.float32)
        zero16 = jnp.zeros((16,), jnp.float32)
        one16 = jnp.ones((16,), jnp.float32)
        zero16i = jnp.zeros((16,), jnp.int32)
        cidx0 = [coli_v[pl.ds(16 * h, 16)] for h in range(2)]
        cidx1 = [coli_v[pl.ds(32 + 16 * h, 16)] for h in range(2)]
        cw0 = [colw_v[pl.ds(16 * h, 16)] for h in range(2)]
        cw1 = [colw_v[pl.ds(32 + 16 * h, 16)] for h in range(2)]

        def body_b(i, c0):
            b = wid * _BPW + i

            pltpu.sync_copy(x_hbm.at[b], rows_v)

            def bct(ct, cc):
                c = ct // _RH
                t = ct % _RH
                w0s = plsc.load_gather(rw_v, [zero16i + t])
                w1s = plsc.load_gather(rw_v, [zero16i + (32 + t)])

                def bq(q, cq):
                    sl = pl.ds(q * 16, 16)
                    ri_v[sl] = (w0s * rows_v[ct, sl]
                                + w1s * rows_v[ct, pl.ds(_Wd + q * 16, 16)])
                    return cq
                lax.fori_loop(0, _Wd // 16, bq, 0)
                obase = c * _PLANE + t * _RSTRIDE
                for h in range(2):
                    g0 = plsc.load_gather(ri_v, [cidx0[h]])
                    g1 = plsc.load_gather(ri_v, [cidx1[h]])
                    inp_v[pl.ds(obase + 16 * h, 16)] = cw0[h] * g0 + cw1[h] * g1
                return cc
            lax.fori_loop(0, _C * _RH, bct, 0)

            def body_v(v, c1):
                def body_k(k, acc):
                    ki = k // 3
                    kj = k % 3
                    obase = i * _OFFS + 2 * k * _OUTS + v * _OW
                    dy = off_v[pl.ds(obase, 16)]
                    dx = off_v[pl.ds(obase + _OUTS, 16)]
                    m = mask_v[pl.ds(i * _MSKS + k * _OUTS + v * _OW, 16)]
                    yy = dy + (v + ki).astype(jnp.float32)
                    xx = dx + kj.astype(jnp.float32) + lanes_f
                    ty = yy.astype(jnp.int32)
                    y0 = ty - jnp.where(ty.astype(jnp.float32) > yy, 1, 0)
                    fy = yy - y0.astype(jnp.float32)
                    tx = xx.astype(jnp.int32)
                    x0 = tx - jnp.where(tx.astype(jnp.float32) > xx, 1, 0)
                    fx = xx - x0.astype(jnp.float32)
                    y1 = y0 + 1
                    x1 = x0 + 1
                    vy0 = jnp.where((y0 >= 0) & (y0 < _RH), one16, zero16)
                    vy1 = jnp.where((y1 >= 0) & (y1 < _RH), one16, zero16)
                    vx0 = jnp.where((x0 >= 0) & (x0 < _RW), one16, zero16)
                    vx1 = jnp.where((x1 >= 0) & (x1 < _RW), one16, zero16)
                    cy0 = jnp.clip(y0, 0, _RH - 1)
                    cy1 = jnp.clip(y1, 0, _RH - 1)
                    cx0 = jnp.clip(x0, 0, _RW - 1)
                    cx1 = jnp.clip(x1, 0, _RW - 1)
                    gy0 = vy0 * (1.0 - fy)
                    gy1 = vy1 * fy
                    gx0 = vx0 * (1.0 - fx)
                    gx1 = vx1 * fx
                    w00 = gy0 * gx0
                    w01 = gy0 * gx1
                    w10 = gy1 * gx0
                    w11 = gy1 * gx1
                    r0 = cy0 * _RSTRIDE
                    r1 = cy1 * _RSTRIDE
                    i00 = r0 + cx0
                    i01 = r0 + cx1
                    i10 = r1 + cx0
                    i11 = r1 + cx1
                    tot = zero16
                    for c in range(_C):
                        base = c * _PLANE
                        v00 = plsc.load_gather(inp_v, [i00 + base])
                        v01 = plsc.load_gather(inp_v, [i01 + base])
                        v10 = plsc.load_gather(inp_v, [i10 + base])
                        v11 = plsc.load_gather(inp_v, [i11 + base])
                        s = w00 * v00 + w01 * v01 + w10 * v10 + w11 * v11
                        wk = plsc.load_gather(
                            w_v, [jnp.full((16,), c * _K, jnp.int32) + k])
                        tot = tot + wk * s
                    return acc + m * tot

                acc = lax.fori_loop(0, _K, body_k, zero16)
                out_v[pl.ds(i * _OUTS + v * _OW, 16)] = acc
                return c1

            lax.fori_loop(0, _OH, body_v, 0)
            return c0

        lax.fori_loop(0, _BPW, body_b, 0)
        pltpu.sync_copy(out_v, out_hbm.at[pl.ds(wid * (_BPW * _OUTS), _BPW * _OUTS)])

    return _fused(xsel, off_flat, mask_flat, w_pad, rw, coli, colw)


def kernel(x, offset, mask, W):
    rw, coli, colw = _consts()
    y0, _, _ = _interp_plan(_RH, _H)
    xsel = jnp.concatenate(
        [x[:, :, int(y0[t]):int(y0[t]) + 2, :] for t in range(_RH)],
        axis=2).reshape(_B, _C * _RH, 2 * _Wd)
    out_flat = _fused_sc(
        xsel,
        offset.reshape(_B * _OFFS),
        mask.reshape(_B * _MSKS),
        jnp.pad(W.reshape(_C * _K), (0, 32 - _C * _K)),
        jnp.asarray(rw),
        jnp.asarray(coli),
        jnp.asarray(colw),
    )
    return out_flat.reshape(_B, 1, _OH, _OW)


# speedup vs baseline: 2.1845x; 1.0459x over previous
import functools

import numpy as np
import jax
import jax.numpy as jnp
from jax import lax
from jax.experimental import pallas as pl
from jax.experimental.pallas import tpu as pltpu
from jax.experimental.pallas import tpu_sc as plsc

_B = 256
_C = 3
_H = 224
_Wd = 224
_RH = 18
_RW = 18
_K = 9
_OH = 16
_OW = 16
_NW = 32
_BPW = _B // _NW
_RSTRIDE = 32
_PLANE = _RH * _RSTRIDE
_IMG = _C * _PLANE
_NROW = 2 * _RH * _C
_NROWP = 112
_OFFS = 2 * _K * _OH * _OW
_MSKS = _K * _OH * _OW
_OUTS = _OH * _OW


def _interp_plan(out_n, in_n):
    s = in_n / out_n
    pos = np.maximum((np.arange(out_n) + 0.5) * s - 0.5, 0.0)
    i0 = np.floor(pos).astype(np.int32)
    i1 = np.minimum(i0 + 1, in_n - 1)
    f = (pos - i0).astype(np.float32)
    return i0, i1, f


def _consts():
    _, _, fy = _interp_plan(_RH, _H)
    rw = np.zeros((64,), np.float32)
    rw[:_RH] = 1.0 - fy
    rw[32:32 + _RH] = fy
    x0, x1, fx = _interp_plan(_RW, _Wd)
    coli = np.zeros((64,), np.int32)
    colw = np.zeros((64,), np.float32)
    coli[:_RW] = x0
    coli[32:32 + _RW] = x1
    colw[:_RW] = 1.0 - fx
    colw[32:32 + _RW] = fx
    return rw, coli, colw


def _fused_sc(xsel, off_flat, mask_flat, w_pad, rw, coli, colw):
    mesh = plsc.VectorSubcoreMesh(core_axis_name="c", subcore_axis_name="s")

    @functools.partial(
        pl.kernel,
        mesh=mesh,
        compiler_params=pltpu.CompilerParams(needs_layout_passes=False),
        out_type=jax.ShapeDtypeStruct((_B * _OUTS,), jnp.float32),
        scratch_types=[
            pltpu.VMEM((_C * _RH, 2 * _Wd), jnp.float32),
            pltpu.VMEM((_Wd,), jnp.float32),
            pltpu.VMEM((_IMG,), jnp.float32),
            pltpu.VMEM((_BPW * _OFFS,), jnp.float32),
            pltpu.VMEM((_BPW * _MSKS,), jnp.float32),
            pltpu.VMEM((32,), jnp.float32),
            pltpu.VMEM((64,), jnp.float32),
            pltpu.VMEM((64,), jnp.int32),
            pltpu.VMEM((64,), jnp.float32),
            pltpu.VMEM((_BPW * _OUTS,), jnp.float32),
            pltpu.SemaphoreType.DMA,
        ],
    )
    def _fused(x_hbm, off_hbm, mask_hbm, w_hbm, rw_hbm,
               coli_hbm, colw_hbm, out_hbm,
               rows_v, ri_v, inp_v, off_v, mask_v, w_v,
               rw_v, coli_v, colw_v, out_v, sem):
        wid = lax.axis_index("s") * 2 + lax.axis_index("c")
        pltpu.sync_copy(off_hbm.at[pl.ds(wid * (_BPW * _OFFS), _BPW * _OFFS)], off_v)
        pltpu.sync_copy(mask_hbm.at[pl.ds(wid * (_BPW * _MSKS), _BPW * _MSKS)], mask_v)
        pltpu.sync_copy(w_hbm, w_v)
        pltpu.sync_copy(rw_hbm, rw_v)
        pltpu.sync_copy(coli_hbm, coli_v)
        pltpu.sync_copy(colw_hbm, colw_v)

        lanes_i = lax.iota(jnp.int32, 16)
        lanes_f = lanes_i.astype(jnp.float32)
        zero16 = jnp.zeros((16,), jnp.float32)
        one16 = jnp.ones((16,), jnp.float32)
        zero16i = jnp.zeros((16,), jnp.int32)
        cidx0 = [coli_v[pl.ds(16 * h, 16)] for h in range(2)]
        cidx1 = [coli_v[pl.ds(32 + 16 * h, 16)] for h in range(2)]
        cw0 = [colw_v[pl.ds(16 * h, 16)] for h in range(2)]
        cw1 = [colw_v[pl.ds(32 + 16 * h, 16)] for h in range(2)]

        def body_b(i, c0):
            b = wid * _BPW + i

            pltpu.sync_copy(x_hbm.at[b], rows_v)

            def bct(ct, cc):
                c = ct // _RH
                t = ct % _RH
                w0s = plsc.load_gather(rw_v, [zero16i + t])
                w1s = plsc.load_gather(rw_v, [zero16i + (32 + t)])
                def bq(q, cq):
                    sl = pl.ds(q * 16, 16)
                    ri_v[sl] = (w0s * rows_v[ct, sl]
                                + w1s * rows_v[ct, pl.ds(_Wd + q * 16, 16)])
                    return cq
                lax.fori_loop(0, _Wd // 16, bq, 0)
                obase = c * _PLANE + t * _RSTRIDE
                for h in range(2):
                    g0 = plsc.load_gather(ri_v, [cidx0[h]])
                    g1 = plsc.load_gather(ri_v, [cidx1[h]])
                    inp_v[pl.ds(obase + 16 * h, 16)] = (cw0[h] * g0
                                                        + cw1[h] * g1)
                return cc
            lax.fori_loop(0, _C * _RH, bct, 0)

            def body_v(v, c1):
                def body_k(k, acc):
                    ki = k // 3
                    kj = k % 3
                    obase = i * _OFFS + 2 * k * _OUTS + v * _OW
                    dy = off_v[pl.ds(obase, 16)]
                    dx = off_v[pl.ds(obase + _OUTS, 16)]
                    m = mask_v[pl.ds(i * _MSKS + k * _OUTS + v * _OW, 16)]
                    yy = dy + (v + ki).astype(jnp.float32)
                    xx = dx + kj.astype(jnp.float32) + lanes_f
                    ty = yy.astype(jnp.int32)
                    y0 = ty - jnp.where(ty.astype(jnp.float32) > yy, 1, 0)
                    fy = yy - y0.astype(jnp.float32)
                    tx = xx.astype(jnp.int32)
                    x0 = tx - jnp.where(tx.astype(jnp.float32) > xx, 1, 0)
                    fx = xx - x0.astype(jnp.float32)
                    y1 = y0 + 1
                    x1 = x0 + 1
                    vy0 = jnp.where((y0 >= 0) & (y0 < _RH), one16, zero16)
                    vy1 = jnp.where((y1 >= 0) & (y1 < _RH), one16, zero16)
                    vx0 = jnp.where((x0 >= 0) & (x0 < _RW), one16, zero16)
                    vx1 = jnp.where((x1 >= 0) & (x1 < _RW), one16, zero16)
                    cy0 = jnp.clip(y0, 0, _RH - 1)
                    cy1 = jnp.clip(y1, 0, _RH - 1)
                    cx0 = jnp.clip(x0, 0, _RW - 1)
                    cx1 = jnp.clip(x1, 0, _RW - 1)
                    gy0 = vy0 * (1.0 - fy)
                    gy1 = vy1 * fy
                    gx0 = vx0 * (1.0 - fx)
                    gx1 = vx1 * fx
                    w00 = gy0 * gx0
                    w01 = gy0 * gx1
                    w10 = gy1 * gx0
                    w11 = gy1 * gx1
                    r0 = cy0 * _RSTRIDE
                    r1 = cy1 * _RSTRIDE
                    i00 = r0 + cx0
                    i01 = r0 + cx1
                    i10 = r1 + cx0
                    i11 = r1 + cx1
                    tot = zero16
                    for c in range(_C):
                        base = c * _PLANE
                        v00 = plsc.load_gather(inp_v, [i00 + base])
                        v01 = plsc.load_gather(inp_v, [i01 + base])
                        v10 = plsc.load_gather(inp_v, [i10 + base])
                        v11 = plsc.load_gather(inp_v, [i11 + base])
                        s = w00 * v00 + w01 * v01 + w10 * v10 + w11 * v11
                        wk = plsc.load_gather(
                            w_v, [jnp.full((16,), c * _K + k, jnp.int32)])
                        tot = tot + wk * s
                    return acc + m * tot

                acc = lax.fori_loop(0, _K, body_k, zero16)
                out_v[pl.ds(i * _OUTS + v * _OW, 16)] = acc
                return c1

            lax.fori_loop(0, _OH, body_v, 0)
            return c0

        lax.fori_loop(0, _BPW, body_b, 0)
        pltpu.sync_copy(out_v, out_hbm.at[pl.ds(wid * (_BPW * _OUTS), _BPW * _OUTS)])

    return _fused(xsel, off_flat, mask_flat, w_pad, rw, coli, colw)


def kernel(x, offset, mask, W):
    rw, coli, colw = _consts()
    y0, y1, _ = _interp_plan(_RH, _H)
    rows36 = jnp.asarray(np.stack([y0, y1], axis=1).reshape(-1))
    xsel = jnp.take(x, rows36, axis=2).reshape(_B, _C * _RH, 2 * _Wd)
    out_flat = _fused_sc(
        xsel,
        offset.reshape(_B * _OFFS),
        mask.reshape(_B * _MSKS),
        jnp.pad(W.reshape(_C * _K), (0, 32 - _C * _K)),
        jnp.asarray(rw),
        jnp.asarray(coli),
        jnp.asarray(colw),
    )
    return out_flat.reshape(_B, 1, _OH, _OW)


# speedup vs baseline: 2.5240x; 1.1554x over previous
import functools

import numpy as np
import jax
import jax.numpy as jnp
from jax import lax
from jax.experimental import pallas as pl
from jax.experimental.pallas import tpu as pltpu
from jax.experimental.pallas import tpu_sc as plsc

_B = 256
_C = 3
_H = 224
_Wd = 224
_RH = 18
_RW = 18
_K = 9
_OH = 16
_OW = 16
_NW = 32
_BPW = _B // _NW
_RSTRIDE = 32
_PLANE = _RH * _RSTRIDE
_IMG = _C * _PLANE
_NROW = 2 * _RH * _C
_NROWP = 112
_OFFS = 2 * _K * _OH * _OW
_MSKS = _K * _OH * _OW
_OUTS = _OH * _OW


def _interp_plan(out_n, in_n):
    s = in_n / out_n
    pos = np.maximum((np.arange(out_n) + 0.5) * s - 0.5, 0.0)
    i0 = np.floor(pos).astype(np.int32)
    i1 = np.minimum(i0 + 1, in_n - 1)
    f = (pos - i0).astype(np.float32)
    return i0, i1, f


def _consts():
    _, _, fy = _interp_plan(_RH, _H)
    rw = np.zeros((64,), np.float32)
    rw[:_RH] = 1.0 - fy
    rw[32:32 + _RH] = fy
    x0, x1, fx = _interp_plan(_RW, _Wd)
    coli = np.zeros((64,), np.int32)
    colw = np.zeros((64,), np.float32)
    coli[:_RW] = x0
    coli[32:32 + _RW] = x1
    colw[:_RW] = 1.0 - fx
    colw[32:32 + _RW] = fx
    return rw, coli, colw


def _fused_sc(xsel, off_flat, mask_flat, w_pad, rw, coli, colw):
    mesh = plsc.VectorSubcoreMesh(core_axis_name="c", subcore_axis_name="s")

    @functools.partial(
        pl.kernel,
        mesh=mesh,
        compiler_params=pltpu.CompilerParams(needs_layout_passes=False),
        out_type=jax.ShapeDtypeStruct((_B * _OUTS,), jnp.float32),
        scratch_types=[
            pltpu.VMEM((_C * _RH * 2 * _Wd,), jnp.float32),
            pltpu.VMEM((_IMG,), jnp.float32),
            pltpu.VMEM((_BPW * _OFFS,), jnp.float32),
            pltpu.VMEM((_BPW * _MSKS,), jnp.float32),
            pltpu.VMEM((32,), jnp.float32),
            pltpu.VMEM((64,), jnp.float32),
            pltpu.VMEM((64,), jnp.int32),
            pltpu.VMEM((64,), jnp.float32),
            pltpu.VMEM((_BPW * _OUTS,), jnp.float32),
            pltpu.SemaphoreType.DMA,
        ],
    )
    def _fused(x_hbm, off_hbm, mask_hbm, w_hbm, rw_hbm,
               coli_hbm, colw_hbm, out_hbm,
               rows_v, inp_v, off_v, mask_v, w_v,
               rw_v, coli_v, colw_v, out_v, sem):
        wid = lax.axis_index("s") * 2 + lax.axis_index("c")
        pltpu.sync_copy(off_hbm.at[pl.ds(wid * (_BPW * _OFFS), _BPW * _OFFS)], off_v)
        pltpu.sync_copy(mask_hbm.at[pl.ds(wid * (_BPW * _MSKS), _BPW * _MSKS)], mask_v)
        pltpu.sync_copy(w_hbm, w_v)
        pltpu.sync_copy(rw_hbm, rw_v)
        pltpu.sync_copy(coli_hbm, coli_v)
        pltpu.sync_copy(colw_hbm, colw_v)

        lanes_i = lax.iota(jnp.int32, 16)
        lanes_f = lanes_i.astype(jnp.float32)
        zero16 = jnp.zeros((16,), jnp.float32)
        one16 = jnp.ones((16,), jnp.float32)
        zero16i = jnp.zeros((16,), jnp.int32)
        cidx0 = [coli_v[pl.ds(16 * h, 16)] for h in range(2)]
        cidx1 = [coli_v[pl.ds(32 + 16 * h, 16)] for h in range(2)]
        cw0 = [colw_v[pl.ds(16 * h, 16)] for h in range(2)]
        cw1 = [colw_v[pl.ds(32 + 16 * h, 16)] for h in range(2)]

        def body_b(i, c0):
            b = wid * _BPW + i

            pltpu.sync_copy(x_hbm.at[pl.ds(b * (2 * _RH * _C * _Wd),
                                           2 * _RH * _C * _Wd)], rows_v)

            def bct(ct, cc):
                c = ct // _RH
                t = ct % _RH
                w0s = plsc.load_gather(rw_v, [zero16i + t])
                w1s = plsc.load_gather(rw_v, [zero16i + (32 + t)])
                base = ct * (2 * _Wd)
                obase = c * _PLANE + t * _RSTRIDE
                for h in range(2):
                    a0 = plsc.load_gather(rows_v, [cidx0[h] + base])
                    a1 = plsc.load_gather(rows_v, [cidx1[h] + base])
                    b0 = plsc.load_gather(rows_v, [cidx0[h] + (base + _Wd)])
                    b1 = plsc.load_gather(rows_v, [cidx1[h] + (base + _Wd)])
                    inp_v[pl.ds(obase + 16 * h, 16)] = (
                        w0s * (cw0[h] * a0 + cw1[h] * a1)
                        + w1s * (cw0[h] * b0 + cw1[h] * b1))
                return cc
            lax.fori_loop(0, _C * _RH, bct, 0)

            def body_v(v, c1):
                def body_k(k, acc):
                    ki = k // 3
                    kj = k % 3
                    obase = i * _OFFS + 2 * k * _OUTS + v * _OW
                    dy = off_v[pl.ds(obase, 16)]
                    dx = off_v[pl.ds(obase + _OUTS, 16)]
                    m = mask_v[pl.ds(i * _MSKS + k * _OUTS + v * _OW, 16)]
                    yy = dy + (v + ki).astype(jnp.float32)
                    xx = dx + kj.astype(jnp.float32) + lanes_f
                    ty = yy.astype(jnp.int32)
                    y0 = ty - jnp.where(ty.astype(jnp.float32) > yy, 1, 0)
                    fy = yy - y0.astype(jnp.float32)
                    tx = xx.astype(jnp.int32)
                    x0 = tx - jnp.where(tx.astype(jnp.float32) > xx, 1, 0)
                    fx = xx - x0.astype(jnp.float32)
                    y1 = y0 + 1
                    x1 = x0 + 1
                    vy0 = jnp.where((y0 >= 0) & (y0 < _RH), one16, zero16)
                    vy1 = jnp.where((y1 >= 0) & (y1 < _RH), one16, zero16)
                    vx0 = jnp.where((x0 >= 0) & (x0 < _RW), one16, zero16)
                    vx1 = jnp.where((x1 >= 0) & (x1 < _RW), one16, zero16)
                    cy0 = jnp.clip(y0, 0, _RH - 1)
                    cy1 = jnp.clip(y1, 0, _RH - 1)
                    cx0 = jnp.clip(x0, 0, _RW - 1)
                    cx1 = jnp.clip(x1, 0, _RW - 1)
                    gy0 = vy0 * (1.0 - fy)
                    gy1 = vy1 * fy
                    gx0 = vx0 * (1.0 - fx)
                    gx1 = vx1 * fx
                    w00 = gy0 * gx0
                    w01 = gy0 * gx1
                    w10 = gy1 * gx0
                    w11 = gy1 * gx1
                    r0 = cy0 * _RSTRIDE
                    r1 = cy1 * _RSTRIDE
                    i00 = r0 + cx0
                    i01 = r0 + cx1
                    i10 = r1 + cx0
                    i11 = r1 + cx1
                    tot = zero16
                    for c in range(_C):
                        base = c * _PLANE
                        v00 = plsc.load_gather(inp_v, [i00 + base])
                        v01 = plsc.load_gather(inp_v, [i01 + base])
                        v10 = plsc.load_gather(inp_v, [i10 + base])
                        v11 = plsc.load_gather(inp_v, [i11 + base])
                        s = w00 * v00 + w01 * v01 + w10 * v10 + w11 * v11
                        wk = plsc.load_gather(
                            w_v, [jnp.full((16,), c * _K + k, jnp.int32)])
                        tot = tot + wk * s
                    return acc + m * tot

                acc = lax.fori_loop(0, _K, body_k, zero16)
                out_v[pl.ds(i * _OUTS + v * _OW, 16)] = acc
                return c1

            lax.fori_loop(0, _OH, body_v, 0)
            return c0

        lax.fori_loop(0, _BPW, body_b, 0)
        pltpu.sync_copy(out_v, out_hbm.at[pl.ds(wid * (_BPW * _OUTS), _BPW * _OUTS)])

    return _fused(xsel, off_flat, mask_flat, w_pad, rw, coli, colw)


def kernel(x, offset, mask, W):
    rw, coli, colw = _consts()
    y0, y1, _ = _interp_plan(_RH, _H)
    rows36 = jnp.asarray(np.stack([y0, y1], axis=1).reshape(-1))
    xsel = jnp.take(x, rows36, axis=2).reshape(_B * _C * _RH * 2 * _Wd)
    out_flat = _fused_sc(
        xsel,
        offset.reshape(_B * _OFFS),
        mask.reshape(_B * _MSKS),
        jnp.pad(W.reshape(_C * _K), (0, 32 - _C * _K)),
        jnp.asarray(rw),
        jnp.asarray(coli),
        jnp.asarray(colw),
    )
    return out_flat.reshape(_B, 1, _OH, _OW)
